# baseline (speedup 1.0000x reference)
import functools

import jax
import jax.numpy as jnp
from jax import lax
from jax.experimental import pallas as pl
from jax.experimental.pallas import tpu as pltpu
from jax.experimental.pallas import tpu_sc as plsc

BATCH = 4096
SEQ = 200
DIM = 64
NUM_WORKERS = 32
RS = 512
BPS = BATCH // RS
GROUPS = SEQ * BPS
GPW = GROUPS // NUM_WORKERS
TPAD = RS + 8

_mesh = plsc.VectorSubcoreMesh(core_axis_name="c", subcore_axis_name="s")


@functools.partial(
    pl.kernel,
    mesh=_mesh,
    compiler_params=pltpu.CompilerParams(use_tc_tiling_on_sc=False, needs_layout_passes=False),
    out_type=jax.ShapeDtypeStruct((SEQ * DIM, BATCH), jnp.float32),
    scratch_types=[
        pltpu.VMEM((RS,), jnp.int32),
        pltpu.VMEM((RS, DIM), jnp.float32),
        pltpu.VMEM((DIM * TPAD,), jnp.float32),
        pltpu.SemaphoreType.DMA,
        pltpu.SemaphoreType.DMA,
    ],
)
def _gather_kernel(idx_hbm, table_hbm, out_hbm, idx_v, g_v, t_v, gsem, ssem):
    wid = lax.axis_index("s") * 2 + lax.axis_index("c")
    g0 = wid * GPW
    lane = lax.iota(jnp.int32, 16)

    def group_body(i, carry):
        g = g0 + i
        s = g // BPS
        b0 = (g % BPS) * RS
        pltpu.sync_copy(idx_hbm.at[pl.ds(s * BATCH + b0, RS)], idx_v)
        for j in range(RS // 16):
            idxv = idx_v[pl.ds(j * 16, 16)]
            pltpu.make_async_copy(
                table_hbm.at[idxv],
                g_v.at[pl.ds(j * 16, 16), :],
                gsem,
            ).start()
        for j in range(RS // 16):
            idxv = idx_v[pl.ds(j * 16, 16)]
            pltpu.make_async_copy(
                table_hbm.at[idxv],
                g_v.at[pl.ds(j * 16, 16), :],
                gsem,
            ).wait()

        def tr_body(t, c2):
            for k in range(DIM // 16):
                v = g_v[t, pl.ds(16 * k, 16)]
                plsc.store_scatter(t_v, [(lane + 16 * k) * TPAD + t], v)
            return c2

        lax.fori_loop(0, RS, tr_body, 0)

        for d in range(DIM):
            pltpu.make_async_copy(
                t_v.at[pl.ds(d * TPAD, RS)],
                out_hbm.at[s * DIM + d, pl.ds(b0, RS)],
                ssem,
            ).start()
        for d in range(DIM):
            pltpu.make_async_copy(
                t_v.at[pl.ds(d * TPAD, RS)],
                out_hbm.at[s * DIM + d, pl.ds(b0, RS)],
                ssem,
            ).wait()
        return carry

    lax.fori_loop(0, GPW, group_body, 0)


def kernel(input, table):
    idx_t = input.T.astype(jnp.int32).reshape(SEQ * BATCH)
    out_t = _gather_kernel(idx_t, table)
    return out_t.reshape(SEQ, DIM, BATCH).transpose(2, 0, 1)


# speedup vs baseline: 1.1087x; 1.1087x over previous
import functools

import jax
import jax.numpy as jnp
from jax import lax
from jax.experimental import pallas as pl
from jax.experimental.pallas import tpu as pltpu
from jax.experimental.pallas import tpu_sc as plsc

BATCH = 4096
SEQ = 200
DIM = 64
NUM_WORKERS = 32
RS = 256
BPS = BATCH // RS
GROUPS = SEQ * BPS
GPW = GROUPS // NUM_WORKERS
TPAD = RS + 8

_mesh = plsc.VectorSubcoreMesh(core_axis_name="c", subcore_axis_name="s")


@functools.partial(
    pl.kernel,
    mesh=_mesh,
    compiler_params=pltpu.CompilerParams(
        use_tc_tiling_on_sc=False, needs_layout_passes=False
    ),
    out_type=jax.ShapeDtypeStruct((SEQ * DIM, BATCH), jnp.float32),
    scratch_types=[
        pltpu.VMEM((GPW * RS,), jnp.int32),
        pltpu.VMEM((RS, DIM), jnp.float32),
        pltpu.VMEM((RS, DIM), jnp.float32),
        pltpu.VMEM((DIM, TPAD), jnp.float32),
        pltpu.VMEM((DIM, TPAD), jnp.float32),
        pltpu.SemaphoreType.DMA,
        pltpu.SemaphoreType.DMA,
        pltpu.SemaphoreType.DMA,
        pltpu.SemaphoreType.DMA,
    ],
)
def _gather_kernel(idx_hbm, table_hbm, out_hbm, idx_all, g_a, g_b, t_a, t_b,
                   gsem_a, gsem_b, ssem_a, ssem_b):
    wid = lax.axis_index("s") * 2 + lax.axis_index("c")
    g0 = wid * GPW
    lane = lax.iota(jnp.int32, 16)

    pltpu.sync_copy(idx_hbm.at[pl.ds(g0 * RS, GPW * RS)], idx_all)

    def fire(li, g_v, gsem):
        for j in range(RS // 16):
            idxv = idx_all[pl.ds(li * RS + j * 16, 16)]
            pltpu.make_async_copy(
                table_hbm.at[idxv], g_v.at[pl.ds(j * 16, 16), :], gsem
            ).start()

    def drain_g(li, g_v, gsem):
        for j in range(RS // 16):
            idxv = idx_all[pl.ds(li * RS + j * 16, 16)]
            pltpu.make_async_copy(
                table_hbm.at[idxv], g_v.at[pl.ds(j * 16, 16), :], gsem
            ).wait()

    def transpose(g_v, t_v):
        def tr8(t8, c2):
            for u in range(8):
                t = t8 * 8 + u
                tv = lane * 0 + t
                for k in range(DIM // 16):
                    v = g_v[t, pl.ds(16 * k, 16)]
                    plsc.store_scatter(t_v, [lane + 16 * k, tv], v)
            return c2

        lax.fori_loop(0, RS // 8, tr8, 0)

    def st_cp(g, t_v, ssem, d):
        s = g // BPS
        b0 = (g % BPS) * RS
        return pltpu.make_async_copy(
            t_v.at[d, pl.ds(0, RS)],
            out_hbm.at[s * DIM + d, pl.ds(b0, RS)],
            ssem,
        )

    def fire_st(g, t_v, ssem):
        for d in range(DIM):
            st_cp(g, t_v, ssem, d).start()

    def drain_st(g, t_v, ssem):
        for d in range(DIM):
            st_cp(g, t_v, ssem, d).wait()

    fire(0, g_a, gsem_a)
    fire(1, g_b, gsem_b)
    drain_g(0, g_a, gsem_a)
    transpose(g_a, t_a)
    fire_st(g0 + 0, t_a, ssem_a)
    fire(2, g_a, gsem_a)
    drain_g(1, g_b, gsem_b)
    transpose(g_b, t_b)
    fire_st(g0 + 1, t_b, ssem_b)
    fire(3, g_b, gsem_b)

    def body(i2, carry):
        li = 2 * i2
        drain_g(li, g_a, gsem_a)
        drain_st(g0 + li, t_a, ssem_a)
        transpose(g_a, t_a)
        fire_st(g0 + li, t_a, ssem_a)
        fire(li + 2, g_a, gsem_a)
        drain_g(li + 1, g_b, gsem_b)
        drain_st(g0 + li + 1, t_b, ssem_b)
        transpose(g_b, t_b)
        fire_st(g0 + li + 1, t_b, ssem_b)
        fire(li + 3, g_b, gsem_b)
        return carry

    lax.fori_loop(1, GPW // 2 - 1, body, 0)

    li = GPW - 2
    drain_g(li, g_a, gsem_a)
    drain_st(g0 + li, t_a, ssem_a)
    transpose(g_a, t_a)
    fire_st(g0 + li, t_a, ssem_a)
    drain_g(li + 1, g_b, gsem_b)
    drain_st(g0 + li + 1, t_b, ssem_b)
    transpose(g_b, t_b)
    fire_st(g0 + li + 1, t_b, ssem_b)
    drain_st(g0 + li, t_a, ssem_a)
    drain_st(g0 + li + 1, t_b, ssem_b)


def kernel(input, table):
    idx_t = input.T.astype(jnp.int32).reshape(SEQ * BATCH)
    out_t = _gather_kernel(idx_t, table)
    return out_t.reshape(SEQ, DIM, BATCH).transpose(2, 0, 1)


# speedup vs baseline: 1.3446x; 1.2128x over previous
import functools

import jax
import jax.numpy as jnp
from jax import lax
from jax.experimental import pallas as pl
from jax.experimental.pallas import tpu as pltpu
from jax.experimental.pallas import tpu_sc as plsc

BATCH = 4096
SEQ = 200
DIM = 64
NUM_WORKERS = 32
RS = 256
BPS = BATCH // RS
GROUPS = SEQ * BPS
GPW = GROUPS // NUM_WORKERS
TPAD = RS + 8

_mesh = plsc.VectorSubcoreMesh(core_axis_name="c", subcore_axis_name="s")


@functools.partial(
    pl.kernel,
    mesh=_mesh,
    compiler_params=pltpu.CompilerParams(
        use_tc_tiling_on_sc=False, needs_layout_passes=False
    ),
    out_type=jax.ShapeDtypeStruct((SEQ * DIM * BATCH // 1024, 8, 128), jnp.float32),
    scratch_types=[
        pltpu.VMEM((GPW * RS,), jnp.int32),
        pltpu.VMEM((RS, DIM), jnp.float32),
        pltpu.VMEM((RS, DIM), jnp.float32),
        pltpu.VMEM((DIM, TPAD), jnp.float32),
        pltpu.VMEM((DIM, TPAD), jnp.float32),
        pltpu.SemaphoreType.DMA,
        pltpu.SemaphoreType.DMA,
        pltpu.SemaphoreType.DMA,
        pltpu.SemaphoreType.DMA,
    ],
)
def _gather_kernel(idx_hbm, table_hbm, out_hbm, idx_all, g_a, g_b, t_a, t_b,
                   gsem_a, gsem_b, ssem_a, ssem_b):
    wid = lax.axis_index("s") * 2 + lax.axis_index("c")
    g0 = wid * GPW
    lane = lax.iota(jnp.int32, 16)

    pltpu.sync_copy(idx_hbm.at[pl.ds(g0 * RS, GPW * RS)], idx_all)

    def fire(li, g_v, gsem):
        for j in range(RS // 16):
            idxv = idx_all[pl.ds(li * RS + j * 16, 16)]
            pltpu.make_async_copy(
                table_hbm.at[idxv], g_v.at[pl.ds(j * 16, 16), :], gsem
            ).start()

    def drain_g(li, g_v, gsem):
        for j in range(RS // 16):
            idxv = idx_all[pl.ds(li * RS + j * 16, 16)]
            pltpu.make_async_copy(
                table_hbm.at[idxv], g_v.at[pl.ds(j * 16, 16), :], gsem
            ).wait()

    def transpose(g_v, t_v):
        def tr8(t8, c2):
            for u in range(8):
                t = t8 * 8 + u
                tv = lane * 0 + t
                for k in range(DIM // 16):
                    v = g_v[t, pl.ds(16 * k, 16)]
                    plsc.store_scatter(t_v, [lane + 16 * k, tv], v)
            return c2

        lax.fori_loop(0, RS // 8, tr8, 0)

    def st_cp(g, t_v, ssem, tr, tc_rel):
        s = g // BPS
        tc0 = (g % BPS) * (RS // 128)
        return pltpu.make_async_copy(
            t_v.at[pl.ds(8 * tr, 8), pl.ds(tc_rel * 128, 128)],
            out_hbm.at[(s * 8 + tr) * 32 + tc0 + tc_rel],
            ssem,
        )

    def fire_st(g, t_v, ssem):
        for tr in range(8):
            for tc_rel in range(RS // 128):
                st_cp(g, t_v, ssem, tr, tc_rel).start()

    def drain_st(g, t_v, ssem):
        for tr in range(8):
            for tc_rel in range(RS // 128):
                st_cp(g, t_v, ssem, tr, tc_rel).wait()

    fire(0, g_a, gsem_a)
    fire(1, g_b, gsem_b)
    drain_g(0, g_a, gsem_a)
    transpose(g_a, t_a)
    fire_st(g0 + 0, t_a, ssem_a)
    fire(2, g_a, gsem_a)
    drain_g(1, g_b, gsem_b)
    transpose(g_b, t_b)
    fire_st(g0 + 1, t_b, ssem_b)
    fire(3, g_b, gsem_b)

    def body(i2, carry):
        li = 2 * i2
        drain_g(li, g_a, gsem_a)
        drain_st(g0 + li, t_a, ssem_a)
        transpose(g_a, t_a)
        fire_st(g0 + li, t_a, ssem_a)
        fire(li + 2, g_a, gsem_a)
        drain_g(li + 1, g_b, gsem_b)
        drain_st(g0 + li + 1, t_b, ssem_b)
        transpose(g_b, t_b)
        fire_st(g0 + li + 1, t_b, ssem_b)
        fire(li + 3, g_b, gsem_b)
        return carry

    lax.fori_loop(1, GPW // 2 - 1, body, 0)

    li = GPW - 2
    drain_g(li, g_a, gsem_a)
    drain_st(g0 + li, t_a, ssem_a)
    transpose(g_a, t_a)
    fire_st(g0 + li, t_a, ssem_a)
    drain_g(li + 1, g_b, gsem_b)
    drain_st(g0 + li + 1, t_b, ssem_b)
    transpose(g_b, t_b)
    fire_st(g0 + li + 1, t_b, ssem_b)
    drain_st(g0 + li, t_a, ssem_a)
    drain_st(g0 + li + 1, t_b, ssem_b)


def kernel(input, table):
    idx_t = input.T.astype(jnp.int32).reshape(SEQ * BATCH)
    out3 = _gather_kernel(idx_t, table)
    out_t = (
        out3.reshape(SEQ, 8, BATCH // 128, 8, 128)
        .transpose(0, 1, 3, 2, 4)
        .reshape(SEQ, DIM, BATCH)
    )
    return out_t.transpose(2, 0, 1)


# speedup vs baseline: 1.4286x; 1.0625x over previous
import functools

import jax
import jax.numpy as jnp
from jax import lax
from jax.experimental import pallas as pl
from jax.experimental.pallas import tpu as pltpu
from jax.experimental.pallas import tpu_sc as plsc

BATCH = 4096
SEQ = 200
DIM = 64
NUM_WORKERS = 32
RS = 256
BPS = BATCH // RS
GROUPS = SEQ * BPS
GPW = GROUPS // NUM_WORKERS
TPAD = RS + 8

_mesh = plsc.VectorSubcoreMesh(core_axis_name="c", subcore_axis_name="s")


@functools.partial(
    pl.kernel,
    mesh=_mesh,
    compiler_params=pltpu.CompilerParams(
        use_tc_tiling_on_sc=False, needs_layout_passes=False
    ),
    out_type=jax.ShapeDtypeStruct((SEQ * DIM * BATCH // 1024, 8, 128), jnp.float32),
    scratch_types=[
        pltpu.VMEM((GPW * RS,), jnp.int32),
        pltpu.VMEM((RS, 128), jnp.float32),
        pltpu.VMEM((RS, 128), jnp.float32),
        pltpu.VMEM((DIM, TPAD), jnp.float32),
        pltpu.VMEM((DIM, TPAD), jnp.float32),
        pltpu.SemaphoreType.DMA,
        pltpu.SemaphoreType.DMA,
        pltpu.SemaphoreType.DMA,
        pltpu.SemaphoreType.DMA,
    ],
)
def _gather_kernel(idx_hbm, table_hbm, out_hbm, idx_all, g_a, g_b, t_a, t_b,
                   gsem_a, gsem_b, ssem_a, ssem_b):
    wid = lax.axis_index("s") * 2 + lax.axis_index("c")
    g0 = wid * GPW
    lane = lax.iota(jnp.int32, 16)

    pltpu.sync_copy(idx_hbm.at[pl.ds(g0 * RS, GPW * RS)], idx_all)

    def fire(li, g_v, gsem):
        for j in range(RS // 16):
            idxv = idx_all[pl.ds(li * RS + j * 16, 16)]
            pltpu.make_async_copy(
                table_hbm.at[idxv], g_v.at[pl.ds(j * 16, 16), :], gsem
            ).start()

    def drain_g(li, g_v, gsem):
        for j in range(RS // 16):
            idxv = idx_all[pl.ds(li * RS + j * 16, 16)]
            pltpu.make_async_copy(
                table_hbm.at[idxv], g_v.at[pl.ds(j * 16, 16), :], gsem
            ).wait()

    def transpose(g_v, t_v):
        def tr8(t8, c2):
            for u in range(8):
                t = t8 * 8 + u
                tv = lane * 0 + t
                for k in range(DIM // 16):
                    v = g_v[t, pl.ds(16 * k, 16)]
                    plsc.store_scatter(t_v, [lane + 16 * k, tv], v)
            return c2

        lax.fori_loop(0, RS // 8, tr8, 0)

    def st_cp(g, t_v, ssem, tr, tc_rel):
        s = g // BPS
        tc0 = (g % BPS) * (RS // 128)
        return pltpu.make_async_copy(
            t_v.at[pl.ds(8 * tr, 8), pl.ds(tc_rel * 128, 128)],
            out_hbm.at[(s * 8 + tr) * 32 + tc0 + tc_rel],
            ssem,
        )

    def fire_st(g, t_v, ssem):
        for tr in range(8):
            for tc_rel in range(RS // 128):
                st_cp(g, t_v, ssem, tr, tc_rel).start()

    def drain_st(g, t_v, ssem):
        for tr in range(8):
            for tc_rel in range(RS // 128):
                st_cp(g, t_v, ssem, tr, tc_rel).wait()

    fire(0, g_a, gsem_a)
    fire(1, g_b, gsem_b)
    drain_g(0, g_a, gsem_a)
    transpose(g_a, t_a)
    fire_st(g0 + 0, t_a, ssem_a)
    fire(2, g_a, gsem_a)
    drain_g(1, g_b, gsem_b)
    transpose(g_b, t_b)
    fire_st(g0 + 1, t_b, ssem_b)
    fire(3, g_b, gsem_b)

    def body(i2, carry):
        li = 2 * i2
        drain_g(li, g_a, gsem_a)
        drain_st(g0 + li, t_a, ssem_a)
        transpose(g_a, t_a)
        fire_st(g0 + li, t_a, ssem_a)
        fire(li + 2, g_a, gsem_a)
        drain_g(li + 1, g_b, gsem_b)
        drain_st(g0 + li + 1, t_b, ssem_b)
        transpose(g_b, t_b)
        fire_st(g0 + li + 1, t_b, ssem_b)
        fire(li + 3, g_b, gsem_b)
        return carry

    lax.fori_loop(1, GPW // 2 - 1, body, 0)

    li = GPW - 2
    drain_g(li, g_a, gsem_a)
    drain_st(g0 + li, t_a, ssem_a)
    transpose(g_a, t_a)
    fire_st(g0 + li, t_a, ssem_a)
    drain_g(li + 1, g_b, gsem_b)
    drain_st(g0 + li + 1, t_b, ssem_b)
    transpose(g_b, t_b)
    fire_st(g0 + li + 1, t_b, ssem_b)
    drain_st(g0 + li, t_a, ssem_a)
    drain_st(g0 + li + 1, t_b, ssem_b)


def kernel(input, table):
    idx_t = input.T.astype(jnp.int32).reshape(SEQ * BATCH)
    table_p = jnp.pad(table, ((0, 0), (0, 128 - DIM)))
    out3 = _gather_kernel(idx_t, table_p)
    out_t = (
        out3.reshape(SEQ, 8, BATCH // 128, 8, 128)
        .transpose(0, 1, 3, 2, 4)
        .reshape(SEQ, DIM, BATCH)
    )
    return out_t.transpose(2, 0, 1)
